# baseline (speedup 1.0000x reference)
import jax
import jax.numpy as jnp
from jax.experimental import pallas as pl
from jax.experimental.pallas import tpu as pltpu


def _leaky(x, s):
    return jnp.where(x >= 0, x, s * x)


def _final_mlp_kernel(agg_ref, wm1_ref, bm1_ref, wm2_ref, bm2_ref, out_ref):
    agg = agg_ref[...]
    u = _leaky(agg @ wm1_ref[...] + bm1_ref[...], 0.12) @ wm2_ref[...] + bm2_ref[...]
    m = jnp.max(u, axis=1, keepdims=True)
    lse = m + jnp.log(jnp.sum(jnp.exp(u - m), axis=1, keepdims=True))
    out_ref[...] = u - lse


def _final_mlp(agg, wm1, bm1, wm2, bm2):
    N = agg.shape[0]
    BN = 2000
    grid = (N // BN,)
    return pl.pallas_call(
        _final_mlp_kernel,
        grid=grid,
        in_specs=[
            pl.BlockSpec((BN, agg.shape[1]), lambda i: (i, 0)),
            pl.BlockSpec((wm1.shape[0], wm1.shape[1]), lambda i: (0, 0)),
            pl.BlockSpec((bm1.shape[0],), lambda i: (0,)),
            pl.BlockSpec((wm2.shape[0], wm2.shape[1]), lambda i: (0, 0)),
            pl.BlockSpec((bm2.shape[0],), lambda i: (0,)),
        ],
        out_specs=pl.BlockSpec((BN, 2), lambda i: (i, 0)),
        out_shape=jax.ShapeDtypeStruct((N, 2), jnp.float32),
    )(agg, wm1, bm1, wm2, bm2)


def kernel(x, params, edge_index):
    N = x.shape[0]
    row = edge_index[0].astype(jnp.int32)
    col = edge_index[1].astype(jnp.int32)

    h = x
    for name in ('ec1', 'ec2', 'ec3'):
        (W1, b1), (W2, b2) = params[name]
        f = h.shape[1]
        A = h @ (W1[:f] - W1[f:]) + b1
        B = h @ W1[f:]
        z = A[col] + B[row]
        he = _leaky(z, 0.1) @ W2 + b2
        agg = jax.ops.segment_max(he, col, num_segments=N)
        h = jnp.where(jnp.isfinite(agg), _leaky(agg, 0.1), 0.0)

    (We1, be1), (We2, be2) = params['edge']
    (Wn1, bn1), (Wn2, bn2) = params['node1']
    (Wm1, bm1), (Wm2, bm2) = params['node2']

    P = h @ We1[:64] + be1
    Q = h @ We1[64:]
    e = _leaky(P[row] + Q[col], 0.12) @ We2 + be2
    R = h @ Wn1[:64] + bn1
    o = _leaky(R[col] + e @ Wn1[64:], 0.12) @ Wn2 + bn2
    cnt = jax.ops.segment_sum(jnp.ones((row.shape[0],), jnp.float32), row, num_segments=N)
    agg = jax.ops.segment_sum(o, row, num_segments=N) / jnp.maximum(cnt, 1.0)[:, None]

    return _final_mlp(agg, Wm1, bm1, Wm2, bm2)


# speedup vs baseline: 1.2240x; 1.2240x over previous
import functools
import jax
import jax.numpy as jnp
from jax import lax
from jax.experimental import pallas as pl
from jax.experimental.pallas import tpu as pltpu
from jax.experimental.pallas import tpu_sc as plsc

NC = 2
NS = 16
NW = NC * NS


def _leaky(x, s):
    return jnp.where(x >= 0, x, s * x)


def _mesh():
    return plsc.VectorSubcoreMesh(core_axis_name="c", subcore_axis_name="s")


def _wid():
    return lax.axis_index("s") * NC + lax.axis_index("c")


def _make_gather_add(E, f, C):
    EW = E // NW
    nch = EW // C

    @functools.partial(
        pl.kernel, mesh=_mesh(),
        compiler_params=pltpu.CompilerParams(
            use_tc_tiling_on_sc=False, needs_layout_passes=False),
        out_type=jax.ShapeDtypeStruct((E, f), jnp.float32),
        scratch_types=[
            pltpu.VMEM((C,), jnp.int32),
            pltpu.VMEM((C,), jnp.int32),
            pltpu.VMEM((C, f), jnp.float32),
            pltpu.VMEM((C, f), jnp.float32),
            pltpu.SemaphoreType.DMA,
            pltpu.SemaphoreType.DMA,
        ],
    )
    def k(a_hbm, b_hbm, col_hbm, row_hbm, z_hbm, colv, rowv, bufa, bufb, sa, sb):
        base = _wid() * EW

        def chunk(j, _):
            off = base + j * C
            pltpu.sync_copy(col_hbm.at[pl.ds(off, C)], colv)
            pltpu.sync_copy(row_hbm.at[pl.ds(off, C)], rowv)
            ca = pltpu.async_copy(a_hbm.at[colv], bufa, sa)
            cb = pltpu.async_copy(b_hbm.at[rowv], bufb, sb)
            ca.wait()
            cb.wait()

            def addrow(i, _):
                for kk in range(f // 16):
                    sl = pl.ds(kk * 16, 16)
                    bufa[i, sl] = bufa[i, sl] + bufb[i, sl]
                return 0

            lax.fori_loop(0, C, addrow, 0)
            pltpu.sync_copy(bufa, z_hbm.at[pl.ds(off, C)])
            return 0

        lax.fori_loop(0, nch, chunk, 0)

    return k


def _make_gather_meta(E, C):
    EW = E // NW
    nch = EW // C

    @functools.partial(
        pl.kernel, mesh=_mesh(),
        compiler_params=pltpu.CompilerParams(
            use_tc_tiling_on_sc=False, needs_layout_passes=False),
        out_type=jax.ShapeDtypeStruct((E, 128), jnp.float32),
        scratch_types=[
            pltpu.VMEM((C,), jnp.int32),
            pltpu.VMEM((C,), jnp.int32),
            pltpu.VMEM((C, 64), jnp.float32),
            pltpu.VMEM((C, 128), jnp.float32),
            pltpu.SemaphoreType.DMA,
            pltpu.SemaphoreType.DMA,
        ],
    )
    def k(p_hbm, qr_hbm, col_hbm, row_hbm, z_hbm, colv, rowv, bufp, bufqr, sa, sb):
        base = _wid() * EW

        def chunk(j, _):
            off = base + j * C
            pltpu.sync_copy(col_hbm.at[pl.ds(off, C)], colv)
            pltpu.sync_copy(row_hbm.at[pl.ds(off, C)], rowv)
            ca = pltpu.async_copy(p_hbm.at[rowv], bufp, sa)
            cb = pltpu.async_copy(qr_hbm.at[colv], bufqr, sb)
            ca.wait()
            cb.wait()

            def addrow(i, _):
                for kk in range(4):
                    sl = pl.ds(kk * 16, 16)
                    bufqr[i, sl] = bufqr[i, sl] + bufp[i, sl]
                return 0

            lax.fori_loop(0, C, addrow, 0)
            pltpu.sync_copy(bufqr, z_hbm.at[pl.ds(off, C)])
            return 0

        lax.fori_loop(0, nch, chunk, 0)

    return k


def _make_scatter(E, N, f, op, C2=2000, F=256):
    NT = N // NW
    NTP = NT + 16
    CAPN = F + C2 + 32
    NCNT = ((NTP + 15) // 16) * 16
    mean = op == 'mean'

    scratch = [
        pltpu.VMEM((NTP, f), jnp.float32),
        pltpu.VMEM((C2,), jnp.int32),
        pltpu.VMEM((CAPN,), jnp.int32),
        pltpu.VMEM((CAPN,), jnp.int32),
        pltpu.VMEM((F, f), jnp.float32),
        pltpu.VMEM((NCNT,), jnp.float32),
        pltpu.SemaphoreType.DMA,
    ]

    @functools.partial(
        pl.kernel, mesh=_mesh(),
        compiler_params=pltpu.CompilerParams(
            use_tc_tiling_on_sc=False, needs_layout_passes=False),
        out_type=jax.ShapeDtypeStruct((N, f), jnp.float32),
        scratch_types=scratch,
    )
    def k(idx_hbm, val_hbm, out_hbm, table, idxv, nbuf, ebuf, vbuf, cnt, sem):
        nbase = _wid() * NT
        lanes = jnp.arange(16, dtype=jnp.int32)
        init = jnp.full((16,), -jnp.inf if not mean else 0.0, jnp.float32)

        def initrow(i, _):
            for kk in range(f // 16):
                table[i, pl.ds(kk * 16, 16)] = init
            return 0

        lax.fori_loop(0, NTP, initrow, 0)

        def initz(i, _):
            ebuf[pl.ds(i * 16, 16)] = jnp.zeros((16,), jnp.int32)
            return 0

        lax.fori_loop(0, CAPN // 16, initz, 0)
        if mean:
            def initc(i, _):
                cnt[pl.ds(i * 16, 16)] = jnp.zeros((16,), jnp.float32)
                return 0
            lax.fori_loop(0, NCNT // 16, initc, 0)

        def rmw_block(q, _):
            n = nbuf[pl.ds(q * 16, 16)]
            rows = lanes + q * 16
            for c in range(f):
                cc = jnp.full((16,), c, jnp.int32)
                vals = plsc.load_gather(vbuf, [rows, cc])
                if mean:
                    plsc.addupdate_scatter(table, [n, cc], vals)
                else:
                    def body(_cnt):
                        cur = plsc.load_gather(table, [n, cc])
                        need = vals > cur
                        plsc.store_scatter(table, [n, cc],
                                           jnp.maximum(cur, vals), mask=need)
                        cur2 = plsc.load_gather(table, [n, cc])
                        return jnp.sum((vals > cur2).astype(jnp.int32))
                    lax.while_loop(lambda cn: cn > 0, body, jnp.int32(1))
            if mean:
                plsc.addupdate_scatter(cnt, [n], jnp.full((16,), 1.0, jnp.float32))
            return 0

        def flush(wp):
            pltpu.async_copy(val_hbm.at[ebuf.at[pl.ds(0, F)]], vbuf, sem).wait()
            lax.fori_loop(0, F // 16, rmw_block, 0)

            def shift(s, _):
                nbuf[pl.ds(s * 16, 16)] = nbuf[pl.ds(F + s * 16, 16)]
                ebuf[pl.ds(s * 16, 16)] = ebuf[pl.ds(F + s * 16, 16)]
                return 0

            lax.fori_loop(0, (CAPN - F) // 16, shift, 0)
            return wp - F

        def chunk(j, wp):
            off = j * C2
            pltpu.sync_copy(idx_hbm.at[pl.ds(off, C2)], idxv)

            def scan16(i, w):
                nv = idxv[pl.ds(i * 16, 16)] - nbase
                msk = (nv >= 0) & (nv < NT)
                ev = lanes + (off + i * 16)
                pc = plsc.cumsum(msk.astype(jnp.int32))
                dest = w + pc - 1
                plsc.store_scatter(nbuf, [dest], nv, mask=msk)
                plsc.store_scatter(ebuf, [dest], ev, mask=msk)
                return w + jnp.sum(msk.astype(jnp.int32))

            wp = lax.fori_loop(0, C2 // 16, scan16, wp)
            return lax.while_loop(lambda w: w >= F, flush, wp)

        wp = lax.fori_loop(0, E // C2, chunk, jnp.int32(0))

        nbuf[pl.ds(wp, 16)] = jnp.full((16,), NT, jnp.int32)
        pltpu.async_copy(val_hbm.at[ebuf.at[pl.ds(0, F)]], vbuf, sem).wait()
        lax.fori_loop(0, (wp + 15) // 16, rmw_block, 0)

        if mean:
            def divg(g, _):
                cl = cnt[pl.ds(g * 16, 16)]
                inv = 1.0 / jnp.maximum(cl, 1.0)
                rows = lanes + g * 16
                for c in range(f):
                    cc = jnp.full((16,), c, jnp.int32)
                    s = plsc.load_gather(table, [rows, cc])
                    plsc.store_scatter(table, [rows, cc], s * inv)
                return 0
            lax.fori_loop(0, (NT + 15) // 16, divg, 0)

        pltpu.sync_copy(table.at[pl.ds(0, NT)], out_hbm.at[pl.ds(nbase, NT)])

    return k


def _node_mm(h, wa, ba, wb, bb, fix_slope=None):
    N, fin = h.shape
    BN = 2000
    ga, gb = wa.shape[1], wb.shape[1]

    def body(h_ref, wa_ref, ba_ref, wb_ref, bb_ref, a_ref, b_ref):
        hv = h_ref[...]
        if fix_slope is not None:
            hv = jnp.where(jnp.isfinite(hv), _leaky(hv, fix_slope), 0.0)
        a_ref[...] = hv @ wa_ref[...] + ba_ref[...]
        b_ref[...] = hv @ wb_ref[...] + bb_ref[...]

    return pl.pallas_call(
        body,
        grid=(N // BN,),
        in_specs=[
            pl.BlockSpec((BN, fin), lambda i: (i, 0)),
            pl.BlockSpec(wa.shape, lambda i: (0, 0)),
            pl.BlockSpec(ba.shape, lambda i: (0,)),
            pl.BlockSpec(wb.shape, lambda i: (0, 0)),
            pl.BlockSpec(bb.shape, lambda i: (0,)),
        ],
        out_specs=[
            pl.BlockSpec((BN, ga), lambda i: (i, 0)),
            pl.BlockSpec((BN, gb), lambda i: (i, 0)),
        ],
        out_shape=[
            jax.ShapeDtypeStruct((N, ga), jnp.float32),
            jax.ShapeDtypeStruct((N, gb), jnp.float32),
        ],
    )(h, wa, ba, wb, bb)


def _edge_mm(z, w2, b2, slope, nout):
    E, f1 = z.shape
    BE = 4000
    f2 = w2.shape[1]
    fo = f2 // nout

    def body(z_ref, w_ref, b_ref, *outs):
        hv = _leaky(z_ref[...], slope) @ w_ref[...] + b_ref[...]
        for i, o_ref in enumerate(outs):
            o_ref[...] = hv[:, i * fo:(i + 1) * fo]

    return pl.pallas_call(
        body,
        grid=(E // BE,),
        in_specs=[
            pl.BlockSpec((BE, f1), lambda i: (i, 0)),
            pl.BlockSpec(w2.shape, lambda i: (0, 0)),
            pl.BlockSpec(b2.shape, lambda i: (0,)),
        ],
        out_specs=[pl.BlockSpec((BE, fo), lambda i: (i, 0))] * nout,
        out_shape=[jax.ShapeDtypeStruct((E, fo), jnp.float32)] * nout,
    )(z, w2, b2)


def _meta_edge_mm(zm, we2, be2, wn1b, wn2, bn2):
    E = zm.shape[0]
    BE = 4000

    def body(z_ref, we2_ref, be2_ref, wn1b_ref, wn2_ref, bn2_ref, o_ref):
        zv = z_ref[...]
        e = _leaky(zv[:, :64], 0.12) @ we2_ref[...] + be2_ref[...]
        o = _leaky(zv[:, 64:] + e @ wn1b_ref[...], 0.12) @ wn2_ref[...] + bn2_ref[...]
        o_ref[...] = o

    return pl.pallas_call(
        body,
        grid=(E // BE,),
        in_specs=[
            pl.BlockSpec((BE, 128), lambda i: (i, 0)),
            pl.BlockSpec(we2.shape, lambda i: (0, 0)),
            pl.BlockSpec(be2.shape, lambda i: (0,)),
            pl.BlockSpec(wn1b.shape, lambda i: (0, 0)),
            pl.BlockSpec(wn2.shape, lambda i: (0, 0)),
            pl.BlockSpec(bn2.shape, lambda i: (0,)),
        ],
        out_specs=pl.BlockSpec((BE, 32), lambda i: (i, 0)),
        out_shape=jax.ShapeDtypeStruct((E, 32), jnp.float32),
    )(zm, we2, be2, wn1b, wn2, bn2)


def _final_mlp(agg, wm1, bm1, wm2, bm2):
    N = agg.shape[0]
    BN = 2000

    def body(agg_ref, wm1_ref, bm1_ref, wm2_ref, bm2_ref, out_ref):
        u = _leaky(agg_ref[...] @ wm1_ref[...] + bm1_ref[...], 0.12) \
            @ wm2_ref[...] + bm2_ref[...]
        m = jnp.max(u, axis=1, keepdims=True)
        lse = m + jnp.log(jnp.sum(jnp.exp(u - m), axis=1, keepdims=True))
        out_ref[...] = u - lse

    return pl.pallas_call(
        body,
        grid=(N // BN,),
        in_specs=[
            pl.BlockSpec((BN, agg.shape[1]), lambda i: (i, 0)),
            pl.BlockSpec(wm1.shape, lambda i: (0, 0)),
            pl.BlockSpec(bm1.shape, lambda i: (0,)),
            pl.BlockSpec(wm2.shape, lambda i: (0, 0)),
            pl.BlockSpec(bm2.shape, lambda i: (0,)),
        ],
        out_specs=pl.BlockSpec((BN, 2), lambda i: (i, 0)),
        out_shape=jax.ShapeDtypeStruct((N, 2), jnp.float32),
    )(agg, wm1, bm1, wm2, bm2)


def kernel(x, params, edge_index):
    N = x.shape[0]
    E = edge_index.shape[1]
    row = edge_index[0].astype(jnp.int32)
    col = edge_index[1].astype(jnp.int32)

    ga32 = _make_gather_add(E, 32, 400)
    ga64 = _make_gather_add(E, 64, 400)
    ga128 = _make_gather_add(E, 128, 400)
    gam = _make_gather_meta(E, 400)
    sm16 = _make_scatter(E, N, 16, 'max')
    sm32 = _make_scatter(E, N, 32, 'max')
    smean = _make_scatter(E, N, 32, 'mean')
    gathers = {32: ga32, 64: ga64, 128: ga128}

    h = x
    slope_prev = None
    for name in ('ec1', 'ec2', 'ec3'):
        (W1, b1), (W2, b2) = params[name]
        f = h.shape[1]
        A, B = _node_mm(h, W1[:f] - W1[f:], b1, W1[f:],
                        jnp.zeros((W1.shape[1],), jnp.float32),
                        fix_slope=slope_prev)
        z = gathers[W1.shape[1]](A, B, col, row)
        if name == 'ec3':
            ha, hb = _edge_mm(z, W2, b2, 0.1, 2)
            h = jnp.concatenate([sm32(col, ha), sm32(col, hb)], axis=1)
        else:
            he, = _edge_mm(z, W2, b2, 0.1, 1)
            h = (sm16 if name == 'ec1' else sm32)(col, he)
        slope_prev = 0.1

    (We1, be1), (We2, be2) = params['edge']
    (Wn1, bn1), (Wn2, bn2) = params['node1']
    (Wm1, bm1), (Wm2, bm2) = params['node2']

    Pm, QR = _node_mm(h, We1[:64], be1,
                      jnp.concatenate([We1[64:], Wn1[:64]], axis=1),
                      jnp.concatenate([jnp.zeros((64,), jnp.float32), bn1]),
                      fix_slope=0.1)
    zm = gam(Pm, QR, col, row)
    o = _meta_edge_mm(zm, We2, be2, Wn1[64:], Wn2, bn2)
    agg = smean(row, o)
    return _final_mlp(agg, Wm1, bm1, Wm2, bm2)


# speedup vs baseline: 1.6021x; 1.3089x over previous
import functools
import jax
import jax.numpy as jnp
from jax import lax
from jax.experimental import pallas as pl
from jax.experimental.pallas import tpu as pltpu
from jax.experimental.pallas import tpu_sc as plsc

NC = 2
NS = 16
NW = NC * NS


def _leaky(x, s):
    return jnp.where(x >= 0, x, s * x)


def _mesh():
    return plsc.VectorSubcoreMesh(core_axis_name="c", subcore_axis_name="s")


def _wid():
    return lax.axis_index("s") * NC + lax.axis_index("c")


def _make_gather_add(E, f, C):
    EW = E // NW
    nch = EW // C

    @functools.partial(
        pl.kernel, mesh=_mesh(),
        compiler_params=pltpu.CompilerParams(
            use_tc_tiling_on_sc=False, needs_layout_passes=False),
        out_type=jax.ShapeDtypeStruct((E, f), jnp.float32),
        scratch_types=[
            pltpu.VMEM((C,), jnp.int32),
            pltpu.VMEM((C,), jnp.int32),
            pltpu.VMEM((C,), jnp.int32),
            pltpu.VMEM((C,), jnp.int32),
            pltpu.VMEM((C, f), jnp.float32),
            pltpu.VMEM((C, f), jnp.float32),
            pltpu.VMEM((C, f), jnp.float32),
            pltpu.VMEM((C, f), jnp.float32),
            pltpu.SemaphoreType.DMA,
            pltpu.SemaphoreType.DMA,
            pltpu.SemaphoreType.DMA,
            pltpu.SemaphoreType.DMA,
        ],
    )
    def k(a_hbm, b_hbm, col_hbm, row_hbm, z_hbm,
          colv0, rowv0, colv1, rowv1, a0, b0, a1, b1, sa0, sb0, sa1, sb1):
        base = _wid() * EW
        colv, rowv = (colv0, colv1), (rowv0, rowv1)
        bufa, bufb = (a0, a1), (b0, b1)
        sa, sb = (sa0, sa1), (sb0, sb1)

        def issue(j, bi):
            off = base + j * C
            pltpu.sync_copy(col_hbm.at[pl.ds(off, C)], colv[bi])
            pltpu.sync_copy(row_hbm.at[pl.ds(off, C)], rowv[bi])
            pltpu.async_copy(a_hbm.at[colv[bi]], bufa[bi], sa[bi])
            pltpu.async_copy(b_hbm.at[rowv[bi]], bufb[bi], sb[bi])

        issue(0, 0)

        def outer(g, _):
            for bi in range(2):
                j = g * 2 + bi
                nb = 1 - bi

                @pl.when(j + 1 < nch)
                def _():
                    issue(j + 1, nb)

                pltpu.make_async_copy(a_hbm.at[colv[bi]], bufa[bi], sa[bi]).wait()
                pltpu.make_async_copy(b_hbm.at[rowv[bi]], bufb[bi], sb[bi]).wait()

                def addrow(i, _):
                    for kk in range(f // 16):
                        sl = pl.ds(kk * 16, 16)
                        bufa[bi][i, sl] = bufa[bi][i, sl] + bufb[bi][i, sl]
                    return 0

                lax.fori_loop(0, C, addrow, 0)
                pltpu.sync_copy(bufa[bi], z_hbm.at[pl.ds(base + j * C, C)])
            return 0

        lax.fori_loop(0, nch // 2, outer, 0)

    return k


def _make_gather_meta(E, C):
    EW = E // NW
    nch = EW // C

    @functools.partial(
        pl.kernel, mesh=_mesh(),
        compiler_params=pltpu.CompilerParams(
            use_tc_tiling_on_sc=False, needs_layout_passes=False),
        out_type=jax.ShapeDtypeStruct((E, 128), jnp.float32),
        scratch_types=[
            pltpu.VMEM((C,), jnp.int32),
            pltpu.VMEM((C,), jnp.int32),
            pltpu.VMEM((C,), jnp.int32),
            pltpu.VMEM((C,), jnp.int32),
            pltpu.VMEM((C, 64), jnp.float32),
            pltpu.VMEM((C, 128), jnp.float32),
            pltpu.VMEM((C, 64), jnp.float32),
            pltpu.VMEM((C, 128), jnp.float32),
            pltpu.SemaphoreType.DMA,
            pltpu.SemaphoreType.DMA,
            pltpu.SemaphoreType.DMA,
            pltpu.SemaphoreType.DMA,
        ],
    )
    def k(p_hbm, qr_hbm, col_hbm, row_hbm, z_hbm,
          colv0, rowv0, colv1, rowv1, p0, qr0, p1, qr1, sa0, sb0, sa1, sb1):
        base = _wid() * EW
        colv, rowv = (colv0, colv1), (rowv0, rowv1)
        bufp, bufqr = (p0, p1), (qr0, qr1)
        sa, sb = (sa0, sa1), (sb0, sb1)

        def issue(j, bi):
            off = base + j * C
            pltpu.sync_copy(col_hbm.at[pl.ds(off, C)], colv[bi])
            pltpu.sync_copy(row_hbm.at[pl.ds(off, C)], rowv[bi])
            pltpu.async_copy(p_hbm.at[rowv[bi]], bufp[bi], sa[bi])
            pltpu.async_copy(qr_hbm.at[colv[bi]], bufqr[bi], sb[bi])

        issue(0, 0)

        def outer(g, _):
            for bi in range(2):
                j = g * 2 + bi
                nb = 1 - bi

                @pl.when(j + 1 < nch)
                def _():
                    issue(j + 1, nb)

                pltpu.make_async_copy(p_hbm.at[rowv[bi]], bufp[bi], sa[bi]).wait()
                pltpu.make_async_copy(qr_hbm.at[colv[bi]], bufqr[bi], sb[bi]).wait()

                def addrow(i, _):
                    for kk in range(4):
                        sl = pl.ds(kk * 16, 16)
                        bufqr[bi][i, sl] = bufqr[bi][i, sl] + bufp[bi][i, sl]
                    return 0

                lax.fori_loop(0, C, addrow, 0)
                pltpu.sync_copy(bufqr[bi], z_hbm.at[pl.ds(base + j * C, C)])
            return 0

        lax.fori_loop(0, nch // 2, outer, 0)

    return k


def _make_scatter(E, N, f, op, C2=2000, F=256):
    NT = N // NW
    NTP = NT + 16
    CAPN = F + C2 + 32
    NCNT = ((NTP + 15) // 16) * 16
    mean = op == 'mean'

    scratch = [
        pltpu.VMEM((NTP, f), jnp.float32),
        pltpu.VMEM((C2,), jnp.int32),
        pltpu.VMEM((CAPN,), jnp.int32),
        pltpu.VMEM((CAPN,), jnp.int32),
        pltpu.VMEM((F, f), jnp.float32),
        pltpu.VMEM((NCNT,), jnp.float32),
        pltpu.SemaphoreType.DMA,
    ]

    @functools.partial(
        pl.kernel, mesh=_mesh(),
        compiler_params=pltpu.CompilerParams(
            use_tc_tiling_on_sc=False, needs_layout_passes=False),
        out_type=jax.ShapeDtypeStruct((N, f), jnp.float32),
        scratch_types=scratch,
    )
    def k(idx_hbm, val_hbm, out_hbm, table, idxv, nbuf, ebuf, vbuf, cnt, sem):
        nbase = _wid() * NT
        lanes = jnp.arange(16, dtype=jnp.int32)
        init = jnp.full((16,), -jnp.inf if not mean else 0.0, jnp.float32)

        def initrow(i, _):
            for kk in range(f // 16):
                table[i, pl.ds(kk * 16, 16)] = init
            return 0

        lax.fori_loop(0, NTP, initrow, 0)

        def initz(i, _):
            ebuf[pl.ds(i * 16, 16)] = jnp.zeros((16,), jnp.int32)
            return 0

        lax.fori_loop(0, CAPN // 16, initz, 0)
        if mean:
            def initc(i, _):
                cnt[pl.ds(i * 16, 16)] = jnp.zeros((16,), jnp.float32)
                return 0
            lax.fori_loop(0, NCNT // 16, initc, 0)

        lanesf = lanes.astype(jnp.float32)

        def rmw_block(q, _):
            n = nbuf[pl.ds(q * 16, 16)]
            rows = lanes + q * 16
            if mean:
                for c in range(f):
                    cc = jnp.full((16,), c, jnp.int32)
                    vals = plsc.load_gather(vbuf, [rows, cc])
                    plsc.addupdate_scatter(table, [n, cc], vals)
                plsc.addupdate_scatter(cnt, [n], jnp.full((16,), 1.0, jnp.float32))
                return 0

            plsc.store_scatter(cnt, [n], lanesf)
            rb = plsc.load_gather(cnt, [n])
            ndup = jnp.sum((rb != lanesf).astype(jnp.int32))

            def fast():
                for c in range(f):
                    cc = jnp.full((16,), c, jnp.int32)
                    vals = plsc.load_gather(vbuf, [rows, cc])
                    cur = plsc.load_gather(table, [n, cc])
                    plsc.store_scatter(table, [n, cc], jnp.maximum(cur, vals))

            def slow():
                for c in range(f):
                    cc = jnp.full((16,), c, jnp.int32)
                    vals = plsc.load_gather(vbuf, [rows, cc])

                    def body(_cnt):
                        cur = plsc.load_gather(table, [n, cc])
                        need = vals > cur
                        plsc.store_scatter(table, [n, cc],
                                           jnp.maximum(cur, vals), mask=need)
                        cur2 = plsc.load_gather(table, [n, cc])
                        return jnp.sum((vals > cur2).astype(jnp.int32))
                    lax.while_loop(lambda cn: cn > 0, body, jnp.int32(1))

            lax.cond(ndup == 0, fast, slow)
            return 0

        def flush(wp):
            pltpu.async_copy(val_hbm.at[ebuf.at[pl.ds(0, F)]], vbuf, sem).wait()
            lax.fori_loop(0, F // 16, rmw_block, 0)

            def shift(s, _):
                nbuf[pl.ds(s * 16, 16)] = nbuf[pl.ds(F + s * 16, 16)]
                ebuf[pl.ds(s * 16, 16)] = ebuf[pl.ds(F + s * 16, 16)]
                return 0

            lax.fori_loop(0, (CAPN - F) // 16, shift, 0)
            return wp - F

        def chunk(j, wp):
            off = j * C2
            pltpu.sync_copy(idx_hbm.at[pl.ds(off, C2)], idxv)

            def scan16(i, wv):
                nv = idxv[pl.ds(i * 16, 16)] - nbase
                msk = (nv >= 0) & (nv < NT)
                ev = lanes + (off + i * 16)
                pc = plsc.cumsum(msk.astype(jnp.int32))
                dest = wv + pc - 1
                plsc.store_scatter(nbuf, [dest], nv, mask=msk)
                plsc.store_scatter(ebuf, [dest], ev, mask=msk)
                return wv + plsc.all_reduce_population_count(msk)

            wv = lax.fori_loop(0, C2 // 16, scan16,
                               jnp.full((16,), wp, jnp.int32))
            wp = jnp.sum(jnp.where(lanes == 0, wv, 0))
            return lax.while_loop(lambda w: w >= F, flush, wp)

        wp = lax.fori_loop(0, E // C2, chunk, jnp.int32(0))

        nbuf[pl.ds(wp, 16)] = jnp.full((16,), NT, jnp.int32)
        pltpu.async_copy(val_hbm.at[ebuf.at[pl.ds(0, F)]], vbuf, sem).wait()
        lax.fori_loop(0, (wp + 15) // 16, rmw_block, 0)

        if mean:
            def divg(g, _):
                cl = cnt[pl.ds(g * 16, 16)]
                inv = 1.0 / jnp.maximum(cl, 1.0)
                rows = lanes + g * 16
                for c in range(f):
                    cc = jnp.full((16,), c, jnp.int32)
                    s = plsc.load_gather(table, [rows, cc])
                    plsc.store_scatter(table, [rows, cc], s * inv)
                return 0
            lax.fori_loop(0, (NT + 15) // 16, divg, 0)

        pltpu.sync_copy(table.at[pl.ds(0, NT)], out_hbm.at[pl.ds(nbase, NT)])

    return k


def _node_mm(h, wa, ba, wb, bb, fix_slope=None):
    N, fin = h.shape
    BN = 2000
    ga, gb = wa.shape[1], wb.shape[1]

    def body(h_ref, wa_ref, ba_ref, wb_ref, bb_ref, a_ref, b_ref):
        hv = h_ref[...]
        if fix_slope is not None:
            hv = jnp.where(jnp.isfinite(hv), _leaky(hv, fix_slope), 0.0)
        a_ref[...] = hv @ wa_ref[...] + ba_ref[...]
        b_ref[...] = hv @ wb_ref[...] + bb_ref[...]

    return pl.pallas_call(
        body,
        grid=(N // BN,),
        in_specs=[
            pl.BlockSpec((BN, fin), lambda i: (i, 0)),
            pl.BlockSpec(wa.shape, lambda i: (0, 0)),
            pl.BlockSpec(ba.shape, lambda i: (0,)),
            pl.BlockSpec(wb.shape, lambda i: (0, 0)),
            pl.BlockSpec(bb.shape, lambda i: (0,)),
        ],
        out_specs=[
            pl.BlockSpec((BN, ga), lambda i: (i, 0)),
            pl.BlockSpec((BN, gb), lambda i: (i, 0)),
        ],
        out_shape=[
            jax.ShapeDtypeStruct((N, ga), jnp.float32),
            jax.ShapeDtypeStruct((N, gb), jnp.float32),
        ],
    )(h, wa, ba, wb, bb)


def _edge_mm(z, w2, b2, slope, nout):
    E, f1 = z.shape
    BE = 4000
    f2 = w2.shape[1]
    fo = f2 // nout

    def body(z_ref, w_ref, b_ref, *outs):
        hv = _leaky(z_ref[...], slope) @ w_ref[...] + b_ref[...]
        for i, o_ref in enumerate(outs):
            o_ref[...] = hv[:, i * fo:(i + 1) * fo]

    return pl.pallas_call(
        body,
        grid=(E // BE,),
        in_specs=[
            pl.BlockSpec((BE, f1), lambda i: (i, 0)),
            pl.BlockSpec(w2.shape, lambda i: (0, 0)),
            pl.BlockSpec(b2.shape, lambda i: (0,)),
        ],
        out_specs=[pl.BlockSpec((BE, fo), lambda i: (i, 0))] * nout,
        out_shape=[jax.ShapeDtypeStruct((E, fo), jnp.float32)] * nout,
    )(z, w2, b2)


def _meta_edge_mm(zm, we2, be2, wn1b, wn2, bn2):
    E = zm.shape[0]
    BE = 4000

    def body(z_ref, we2_ref, be2_ref, wn1b_ref, wn2_ref, bn2_ref, o_ref):
        zv = z_ref[...]
        e = _leaky(zv[:, :64], 0.12) @ we2_ref[...] + be2_ref[...]
        o = _leaky(zv[:, 64:] + e @ wn1b_ref[...], 0.12) @ wn2_ref[...] + bn2_ref[...]
        o_ref[...] = o

    return pl.pallas_call(
        body,
        grid=(E // BE,),
        in_specs=[
            pl.BlockSpec((BE, 128), lambda i: (i, 0)),
            pl.BlockSpec(we2.shape, lambda i: (0, 0)),
            pl.BlockSpec(be2.shape, lambda i: (0,)),
            pl.BlockSpec(wn1b.shape, lambda i: (0, 0)),
            pl.BlockSpec(wn2.shape, lambda i: (0, 0)),
            pl.BlockSpec(bn2.shape, lambda i: (0,)),
        ],
        out_specs=pl.BlockSpec((BE, 32), lambda i: (i, 0)),
        out_shape=jax.ShapeDtypeStruct((E, 32), jnp.float32),
    )(zm, we2, be2, wn1b, wn2, bn2)


def _final_mlp(agg, wm1, bm1, wm2, bm2):
    N = agg.shape[0]
    BN = 2000

    def body(agg_ref, wm1_ref, bm1_ref, wm2_ref, bm2_ref, out_ref):
        u = _leaky(agg_ref[...] @ wm1_ref[...] + bm1_ref[...], 0.12) \
            @ wm2_ref[...] + bm2_ref[...]
        m = jnp.max(u, axis=1, keepdims=True)
        lse = m + jnp.log(jnp.sum(jnp.exp(u - m), axis=1, keepdims=True))
        out_ref[...] = u - lse

    return pl.pallas_call(
        body,
        grid=(N // BN,),
        in_specs=[
            pl.BlockSpec((BN, agg.shape[1]), lambda i: (i, 0)),
            pl.BlockSpec(wm1.shape, lambda i: (0, 0)),
            pl.BlockSpec(bm1.shape, lambda i: (0,)),
            pl.BlockSpec(wm2.shape, lambda i: (0, 0)),
            pl.BlockSpec(bm2.shape, lambda i: (0,)),
        ],
        out_specs=pl.BlockSpec((BN, 2), lambda i: (i, 0)),
        out_shape=jax.ShapeDtypeStruct((N, 2), jnp.float32),
    )(agg, wm1, bm1, wm2, bm2)


def kernel(x, params, edge_index):
    N = x.shape[0]
    E = edge_index.shape[1]
    row = edge_index[0].astype(jnp.int32)
    col = edge_index[1].astype(jnp.int32)

    ga32 = _make_gather_add(E, 32, 200)
    ga64 = _make_gather_add(E, 64, 200)
    ga128 = _make_gather_add(E, 128, 200)
    gam = _make_gather_meta(E, 200)
    sm16 = _make_scatter(E, N, 16, 'max')
    sm32 = _make_scatter(E, N, 32, 'max')
    smean = _make_scatter(E, N, 32, 'mean')
    gathers = {32: ga32, 64: ga64, 128: ga128}

    h = x
    slope_prev = None
    for name in ('ec1', 'ec2', 'ec3'):
        (W1, b1), (W2, b2) = params[name]
        f = h.shape[1]
        A, B = _node_mm(h, W1[:f] - W1[f:], b1, W1[f:],
                        jnp.zeros((W1.shape[1],), jnp.float32),
                        fix_slope=slope_prev)
        z = gathers[W1.shape[1]](A, B, col, row)
        if name == 'ec3':
            ha, hb = _edge_mm(z, W2, b2, 0.1, 2)
            h = jnp.concatenate([sm32(col, ha), sm32(col, hb)], axis=1)
        else:
            he, = _edge_mm(z, W2, b2, 0.1, 1)
            h = (sm16 if name == 'ec1' else sm32)(col, he)
        slope_prev = 0.1

    (We1, be1), (We2, be2) = params['edge']
    (Wn1, bn1), (Wn2, bn2) = params['node1']
    (Wm1, bm1), (Wm2, bm2) = params['node2']

    Pm, QR = _node_mm(h, We1[:64], be1,
                      jnp.concatenate([We1[64:], Wn1[:64]], axis=1),
                      jnp.concatenate([jnp.zeros((64,), jnp.float32), bn1]),
                      fix_slope=0.1)
    zm = gam(Pm, QR, col, row)
    o = _meta_edge_mm(zm, We2, be2, Wn1[64:], Wn2, bn2)
    agg = smean(row, o)
    return _final_mlp(agg, Wm1, bm1, Wm2, bm2)
